# baseline (speedup 1.0000x reference)
import functools

import jax
import jax.numpy as jnp
from jax import lax
from jax.experimental import pallas as pl
from jax.experimental.pallas import tpu as pltpu
from jax.experimental.pallas import tpu_sc as plsc

E = 8
K = 2
T = 2048
D = 768
F = 1024
BT = 256
NB = K * T // BT + E
NP = NB * BT
NROW = 32
PC = K * T // NROW
WREP = 128


def _gate_dispatch_body(x_ref, wg_ref, pos_ref, wk_ref, be_ref):
    xg = x_ref[...]
    logits = lax.dot_general(xg, wg_ref[...], (((1,), (0,)), ((), ())),
                             preferred_element_type=jnp.float32)
    ei = lax.broadcasted_iota(jnp.int32, (T, E), 1)
    m0 = jnp.max(logits, axis=1, keepdims=True)
    i0 = jnp.min(jnp.where(logits == m0, ei, E), axis=1, keepdims=True)
    l2 = jnp.where(ei == i0, -jnp.inf, logits)
    m1 = jnp.max(l2, axis=1, keepdims=True)
    i1 = jnp.min(jnp.where(l2 == m1, ei, E), axis=1, keepdims=True)
    d = jnp.exp(m1 - m0)
    w0 = 1.0 / (1.0 + d)
    w1 = d / (1.0 + d)

    eq = jnp.concatenate([i0, i1], axis=0)
    ohi = lax.broadcasted_iota(jnp.int32, (K * T, E), 1)
    oh = (eq == ohi).astype(jnp.float32)
    tri = (lax.broadcasted_iota(jnp.int32, (PC, PC), 0)
           > lax.broadcasted_iota(jnp.int32, (PC, PC), 1)).astype(jnp.float32)
    carry = jnp.zeros((1, E), jnp.float32)
    parts = []
    for g in range(NROW):
        blk = oh[g * PC:(g + 1) * PC, :]
        wexcl = lax.dot_general(tri, blk, (((1,), (0,)), ((), ())),
                                preferred_element_type=jnp.float32)
        parts.append(wexcl + carry)
        carry = carry + jnp.sum(blk, axis=0, keepdims=True)
    excl = jnp.concatenate(parts, axis=0)

    counts = carry.astype(jnp.int32)
    pcounts = ((counts + (BT - 1)) // BT) * BT
    tri8 = (lax.broadcasted_iota(jnp.int32, (E, E), 0)
            < lax.broadcasted_iota(jnp.int32, (E, E), 1)).astype(jnp.float32)
    po = lax.dot_general(pcounts.astype(jnp.float32), tri8,
                         (((1,), (0,)), ((), ())),
                         preferred_element_type=jnp.float32)
    rank = jnp.sum(excl * oh, axis=1, keepdims=True)
    offs = lax.dot_general(oh, po.reshape(E, 1), (((1,), (0,)), ((), ())),
                           preferred_element_type=jnp.float32)
    pos = (rank + offs).astype(jnp.int32)
    pos_ref[...] = pos.reshape(NROW, PC)

    wq = jnp.concatenate([w0, w1], axis=0).reshape(NROW, PC, 1)
    wk_ref[...] = jnp.broadcast_to(wq, (NROW, PC, WREP))

    bstart = lax.broadcasted_iota(jnp.int32, (NB, E), 0) * BT
    cmp = (bstart >= po.astype(jnp.int32)).astype(jnp.int32)
    be_ref[...] = jnp.sum(cmp, axis=1, keepdims=True) - 1


def _gate_dispatch(x2d, wg, interpret=False):
    return pl.pallas_call(
        _gate_dispatch_body,
        out_shape=[
            jax.ShapeDtypeStruct((NROW, PC), jnp.int32),
            jax.ShapeDtypeStruct((NROW, PC, WREP), jnp.float32),
            jax.ShapeDtypeStruct((NB, 1), jnp.int32),
        ],
        interpret=interpret,
    )(x2d, wg)


def _expert_mlp_body(be_ref, xs_ref, w1_ref, b1_ref, w2_ref, b2_ref, rw_ref,
                     ys_ref):
    h = lax.dot_general(xs_ref[...], w1_ref[0], (((1,), (0,)), ((), ())),
                        preferred_element_type=jnp.float32)
    h = h + b1_ref[0]
    h = 0.5 * h * (1.0 + lax.erf(h * 0.7071067811865476))
    y = lax.dot_general(h, w2_ref[0], (((1,), (0,)), ((), ())),
                        preferred_element_type=jnp.float32)
    y = y + b2_ref[0]
    ys_ref[...] = y * rw_ref[:, :1]


def _expert_mlp(be, xs, w1, b1, w2, b2, rw, interpret=False):
    grid_spec = pltpu.PrefetchScalarGridSpec(
        num_scalar_prefetch=1,
        grid=(NB,),
        in_specs=[
            pl.BlockSpec((BT, D), lambda b, be: (b, 0)),
            pl.BlockSpec((1, D, F), lambda b, be: (be[b], 0, 0)),
            pl.BlockSpec((1, 1, F), lambda b, be: (be[b], 0, 0)),
            pl.BlockSpec((1, F, D), lambda b, be: (be[b], 0, 0)),
            pl.BlockSpec((1, 1, D), lambda b, be: (be[b], 0, 0)),
            pl.BlockSpec((BT, WREP), lambda b, be: (b, 0)),
        ],
        out_specs=pl.BlockSpec((BT, D), lambda b, be: (b, 0)),
    )
    return pl.pallas_call(
        _expert_mlp_body,
        grid_spec=grid_spec,
        out_shape=jax.ShapeDtypeStruct((NP, D), jnp.float32),
        compiler_params=pltpu.CompilerParams(
            dimension_semantics=("parallel",)),
        interpret=interpret,
    )(be, xs, w1, b1.reshape(E, 1, F), w2, b2.reshape(E, 1, D), rw)


def _sc_mesh():
    return plsc.VectorSubcoreMesh(core_axis_name="c", subcore_axis_name="s")


def _scatter_dispatch(x2d, posr, wkr):
    @functools.partial(
        pl.kernel,
        mesh=_sc_mesh(),
        out_type=[
            jax.ShapeDtypeStruct((NP, D), jnp.float32),
            jax.ShapeDtypeStruct((NP, WREP), jnp.float32),
        ],
        scratch_types=[
            pltpu.VMEM((PC, D), jnp.float32),
            pltpu.VMEM((PC,), jnp.int32),
            pltpu.VMEM((PC, WREP), jnp.float32),
            pltpu.SemaphoreType.DMA,
            pltpu.SemaphoreType.DMA,
        ],
    )
    def k(x_hbm, pos_hbm, wk_hbm, xs_hbm, rw_hbm, xbuf, idx_v, wbuf, s1, s2):
        r = lax.axis_index("s") * 2 + lax.axis_index("c")
        t0 = (r % (NROW // K)) * PC
        pltpu.sync_copy(x_hbm.at[pl.ds(t0, PC)], xbuf)
        pltpu.sync_copy(pos_hbm.at[r], idx_v)
        pltpu.sync_copy(wk_hbm.at[r], wbuf)
        cp1 = pltpu.async_copy(xbuf, xs_hbm.at[idx_v], s1)
        cp2 = pltpu.async_copy(wbuf, rw_hbm.at[idx_v], s2)
        cp1.wait()
        cp2.wait()

    return k(x2d, posr, wkr)


def _combine(ys, posr):
    tpw = T // 32

    @functools.partial(
        pl.kernel,
        mesh=_sc_mesh(),
        out_type=jax.ShapeDtypeStruct((T, D), jnp.float32),
        scratch_types=[
            pltpu.VMEM((tpw, D), jnp.float32),
            pltpu.VMEM((tpw, D), jnp.float32),
            pltpu.VMEM((tpw,), jnp.int32),
            pltpu.VMEM((tpw,), jnp.int32),
            pltpu.SemaphoreType.DMA,
        ],
    )
    def k(ys_hbm, pos_hbm, out_hbm, a0, a1, idx0, idx1, sem):
        w = lax.axis_index("s") * 2 + lax.axis_index("c")
        tok0 = w * tpw
        r0 = w // 2
        off = (w % 2) * tpw
        pltpu.sync_copy(pos_hbm.at[r0, pl.ds(off, tpw)], idx0)
        pltpu.sync_copy(pos_hbm.at[r0 + NROW // K, pl.ds(off, tpw)], idx1)
        cp0 = pltpu.async_copy(ys_hbm.at[idx0], a0, sem)
        cp1 = pltpu.async_copy(ys_hbm.at[idx1], a1, sem)
        cp0.wait()
        cp1.wait()

        def row(i, _):
            for j in range(D // 16):
                cs = pl.ds(j * 16, 16)
                a0[i, cs] = a0[i, cs] + a1[i, cs]
            return 0
        lax.fori_loop(0, tpw, row, 0)
        pltpu.sync_copy(a0, out_hbm.at[pl.ds(tok0, tpw)])

    return k(ys, posr)


def kernel(x, Wg, W1, b1, W2, b2):
    x2d = x.reshape(T, D)
    posr, wkr, be = _gate_dispatch(x2d, Wg)
    xs, rw = _scatter_dispatch(x2d, posr, wkr)
    ys = _expert_mlp(be.reshape(NB), xs, W1, b1, W2, b2, rw)
    out = _combine(ys, posr)
    return out.reshape(1, T, D)


# speedup vs baseline: 1.0801x; 1.0801x over previous
import functools

import jax
import jax.numpy as jnp
from jax import lax
from jax.experimental import pallas as pl
from jax.experimental.pallas import tpu as pltpu
from jax.experimental.pallas import tpu_sc as plsc

E = 8
K = 2
T = 2048
D = 768
F = 1024
BT = 256
NB = K * T // BT + E
NP = NB * BT
NROW = 32
PC = K * T // NROW
WREP = 128


def _gate_dispatch_body(x_ref, wg_ref, pos_ref, wk_ref, pf_ref):
    xg = x_ref[...]
    logits = lax.dot_general(xg, wg_ref[...], (((1,), (0,)), ((), ())),
                             preferred_element_type=jnp.float32)
    ei = lax.broadcasted_iota(jnp.int32, (T, E), 1)
    m0 = jnp.max(logits, axis=1, keepdims=True)
    i0 = jnp.min(jnp.where(logits == m0, ei, E), axis=1, keepdims=True)
    l2 = jnp.where(ei == i0, -jnp.inf, logits)
    m1 = jnp.max(l2, axis=1, keepdims=True)
    i1 = jnp.min(jnp.where(l2 == m1, ei, E), axis=1, keepdims=True)
    d = jnp.exp(m1 - m0)
    w0 = 1.0 / (1.0 + d)
    w1 = d / (1.0 + d)

    eq = jnp.concatenate([i0, i1], axis=0)
    ohi = lax.broadcasted_iota(jnp.int32, (K * T, E), 1)
    oh = (eq == ohi).astype(jnp.float32)
    tri = (lax.broadcasted_iota(jnp.int32, (PC, PC), 0)
           > lax.broadcasted_iota(jnp.int32, (PC, PC), 1)).astype(jnp.float32)
    carry = jnp.zeros((1, E), jnp.float32)
    parts = []
    for g in range(NROW):
        blk = oh[g * PC:(g + 1) * PC, :]
        wexcl = lax.dot_general(tri, blk, (((1,), (0,)), ((), ())),
                                preferred_element_type=jnp.float32)
        parts.append(wexcl + carry)
        carry = carry + jnp.sum(blk, axis=0, keepdims=True)
    excl = jnp.concatenate(parts, axis=0)

    counts = carry.astype(jnp.int32)
    pcounts = ((counts + (BT - 1)) // BT) * BT
    tri8 = (lax.broadcasted_iota(jnp.int32, (E, E), 0)
            < lax.broadcasted_iota(jnp.int32, (E, E), 1)).astype(jnp.float32)
    po = lax.dot_general(pcounts.astype(jnp.float32), tri8,
                         (((1,), (0,)), ((), ())),
                         preferred_element_type=jnp.float32)
    rank = jnp.sum(excl * oh, axis=1, keepdims=True)
    offs = lax.dot_general(oh, po.reshape(E, 1), (((1,), (0,)), ((), ())),
                           preferred_element_type=jnp.float32)
    pos = (rank + offs).astype(jnp.int32)
    pos_ref[...] = pos.reshape(NROW, PC)

    wq = jnp.concatenate([w0, w1], axis=0).reshape(NROW, PC, 1)
    wk_ref[...] = jnp.broadcast_to(wq, (NROW, PC, WREP))

    po_col = po.astype(jnp.int32).reshape(E, 1)
    biota = lax.broadcasted_iota(jnp.int32, (E, NB), 1) * BT
    beT = jnp.sum((biota >= po_col).astype(jnp.int32), axis=0, keepdims=True) - 1
    prevT = jnp.concatenate(
        [jnp.full((1, 1), -1, jnp.int32), beT[:, :-1]], axis=1)
    isfT = (beT != prevT).astype(jnp.int32)
    triNB = (lax.broadcasted_iota(jnp.int32, (NB, NB), 0)
             <= lax.broadcasted_iota(jnp.int32, (NB, NB), 1)).astype(jnp.float32)
    segT = lax.dot_general(isfT.astype(jnp.float32), triNB,
                           (((1,), (0,)), ((), ())),
                           preferred_element_type=jnp.float32).astype(jnp.int32) - 1
    slT = segT % 2
    beB = jnp.broadcast_to(beT, (NB, NB))
    beC = beT.reshape(NB, 1)
    q = jnp.where(beB > beC, beB, E)
    nxtT = jnp.min(q, axis=1, keepdims=True).reshape(1, NB)
    hnT = (nxtT < E).astype(jnp.int32)
    nxtT = jnp.minimum(nxtT, E - 1)
    pf_ref[...] = jnp.concatenate([beT, isfT, slT, nxtT, hnT], axis=0)


def _gate_dispatch(x2d, wg, interpret=False):
    return pl.pallas_call(
        _gate_dispatch_body,
        out_shape=[
            jax.ShapeDtypeStruct((NROW, PC), jnp.int32),
            jax.ShapeDtypeStruct((NROW, PC, WREP), jnp.float32),
            jax.ShapeDtypeStruct((5, NB), jnp.int32),
        ],
        interpret=interpret,
    )(x2d, wg)


def _expert_mlp_body(pf_ref, xs_ref, w1_hbm, b1_ref, w2_hbm, b2_ref, rw_ref,
                     ys_ref, w1b, w2b, sems):
    b = pl.program_id(0)
    e = pf_ref[0, b]
    isf = pf_ref[1, b]
    s = pf_ref[2, b]
    ne = pf_ref[3, b]
    hn = pf_ref[4, b]

    def start_copy(slot, expert):
        pltpu.make_async_copy(w1_hbm.at[expert], w1b.at[slot],
                              sems.at[slot, 0]).start()
        pltpu.make_async_copy(w2_hbm.at[expert], w2b.at[slot],
                              sems.at[slot, 1]).start()

    @pl.when(b == 0)
    def _():
        start_copy(s, e)

    @pl.when(isf == 1)
    def _():
        pltpu.make_async_copy(w1_hbm.at[e], w1b.at[s], sems.at[s, 0]).wait()
        pltpu.make_async_copy(w2_hbm.at[e], w2b.at[s], sems.at[s, 1]).wait()

        @pl.when(hn == 1)
        def _():
            start_copy(1 - s, ne)

    h = lax.dot_general(xs_ref[...], w1b[s], (((1,), (0,)), ((), ())),
                        preferred_element_type=jnp.float32)
    h = h + b1_ref[e]
    h = 0.5 * h * (1.0 + lax.erf(h * 0.7071067811865476))
    y = lax.dot_general(h, w2b[s], (((1,), (0,)), ((), ())),
                        preferred_element_type=jnp.float32)
    y = y + b2_ref[e]
    ys_ref[...] = y * rw_ref[:, :1]


def _expert_mlp(pf, xs, w1, b1, w2, b2, rw, interpret=False):
    grid_spec = pltpu.PrefetchScalarGridSpec(
        num_scalar_prefetch=1,
        grid=(NB,),
        in_specs=[
            pl.BlockSpec((BT, D), lambda b, pf: (b, 0)),
            pl.BlockSpec(memory_space=pltpu.HBM),
            pl.BlockSpec((E, 1, F), lambda b, pf: (0, 0, 0)),
            pl.BlockSpec(memory_space=pltpu.HBM),
            pl.BlockSpec((E, 1, D), lambda b, pf: (0, 0, 0)),
            pl.BlockSpec((BT, WREP), lambda b, pf: (b, 0)),
        ],
        out_specs=pl.BlockSpec((BT, D), lambda b, pf: (b, 0)),
        scratch_shapes=[
            pltpu.VMEM((2, D, F), jnp.float32),
            pltpu.VMEM((2, F, D), jnp.float32),
            pltpu.SemaphoreType.DMA((2, 2)),
        ],
    )
    return pl.pallas_call(
        _expert_mlp_body,
        grid_spec=grid_spec,
        out_shape=jax.ShapeDtypeStruct((NP, D), jnp.float32),
        compiler_params=pltpu.CompilerParams(
            dimension_semantics=("arbitrary",)),
        interpret=interpret,
    )(pf, xs, w1, b1.reshape(E, 1, F), w2, b2.reshape(E, 1, D), rw)


def _sc_mesh():
    return plsc.VectorSubcoreMesh(core_axis_name="c", subcore_axis_name="s")


def _scatter_dispatch(x2d, posr, wkr):
    @functools.partial(
        pl.kernel,
        mesh=_sc_mesh(),
        out_type=[
            jax.ShapeDtypeStruct((NP, D), jnp.float32),
            jax.ShapeDtypeStruct((NP, WREP), jnp.float32),
        ],
        scratch_types=[
            pltpu.VMEM((PC, D), jnp.float32),
            pltpu.VMEM((PC,), jnp.int32),
            pltpu.VMEM((PC, WREP), jnp.float32),
            pltpu.SemaphoreType.DMA,
            pltpu.SemaphoreType.DMA,
        ],
    )
    def k(x_hbm, pos_hbm, wk_hbm, xs_hbm, rw_hbm, xbuf, idx_v, wbuf, s1, s2):
        r = lax.axis_index("s") * 2 + lax.axis_index("c")
        t0 = (r % (NROW // K)) * PC
        pltpu.sync_copy(x_hbm.at[pl.ds(t0, PC)], xbuf)
        pltpu.sync_copy(pos_hbm.at[r], idx_v)
        pltpu.sync_copy(wk_hbm.at[r], wbuf)
        cp1 = pltpu.async_copy(xbuf, xs_hbm.at[idx_v], s1)
        cp2 = pltpu.async_copy(wbuf, rw_hbm.at[idx_v], s2)
        cp1.wait()
        cp2.wait()

    return k(x2d, posr, wkr)


def _combine(ys, posr):
    tpw = T // 32

    @functools.partial(
        pl.kernel,
        mesh=_sc_mesh(),
        out_type=jax.ShapeDtypeStruct((T, D), jnp.float32),
        scratch_types=[
            pltpu.VMEM((tpw, D), jnp.float32),
            pltpu.VMEM((tpw, D), jnp.float32),
            pltpu.VMEM((tpw,), jnp.int32),
            pltpu.VMEM((tpw,), jnp.int32),
            pltpu.SemaphoreType.DMA,
        ],
    )
    def k(ys_hbm, pos_hbm, out_hbm, a0, a1, idx0, idx1, sem):
        w = lax.axis_index("s") * 2 + lax.axis_index("c")
        tok0 = w * tpw
        r0 = w // 2
        off = (w % 2) * tpw
        pltpu.sync_copy(pos_hbm.at[r0, pl.ds(off, tpw)], idx0)
        pltpu.sync_copy(pos_hbm.at[r0 + NROW // K, pl.ds(off, tpw)], idx1)
        cp0 = pltpu.async_copy(ys_hbm.at[idx0], a0, sem)
        cp1 = pltpu.async_copy(ys_hbm.at[idx1], a1, sem)
        cp0.wait()
        cp1.wait()

        def row(i, _):
            for j in range(D // 16):
                cs = pl.ds(j * 16, 16)
                a0[i, cs] = a0[i, cs] + a1[i, cs]
            return 0
        lax.fori_loop(0, tpw, row, 0)
        pltpu.sync_copy(a0, out_hbm.at[pl.ds(tok0, tpw)])

    return k(ys, posr)


def kernel(x, Wg, W1, b1, W2, b2):
    x2d = x.reshape(T, D)
    posr, wkr, pf = _gate_dispatch(x2d, Wg)
    xs, rw = _scatter_dispatch(x2d, posr, wkr)
    ys = _expert_mlp(pf, xs, W1, b1, W2, b2, rw)
    out = _combine(ys, posr)
    return out.reshape(1, T, D)


# speedup vs baseline: 1.0849x; 1.0045x over previous
import functools

import jax
import jax.numpy as jnp
from jax import lax
from jax.experimental import pallas as pl
from jax.experimental.pallas import tpu as pltpu
from jax.experimental.pallas import tpu_sc as plsc

E = 8
K = 2
T = 2048
D = 768
F = 1024
BT = 256
NB = K * T // BT + E
NP = NB * BT
NROW = 32
PC = K * T // NROW
WREP = 128


def _gate_dispatch_body(x_ref, wg_ref, pos_ref, wk_ref, pf_ref):
    xg = x_ref[...]
    logits = lax.dot_general(xg, wg_ref[...], (((1,), (0,)), ((), ())),
                             preferred_element_type=jnp.float32)
    ei = lax.broadcasted_iota(jnp.int32, (T, E), 1)
    m0 = jnp.max(logits, axis=1, keepdims=True)
    i0 = jnp.min(jnp.where(logits == m0, ei, E), axis=1, keepdims=True)
    l2 = jnp.where(ei == i0, -jnp.inf, logits)
    m1 = jnp.max(l2, axis=1, keepdims=True)
    i1 = jnp.min(jnp.where(l2 == m1, ei, E), axis=1, keepdims=True)
    d = jnp.exp(m1 - m0)
    w0 = 1.0 / (1.0 + d)
    w1 = d / (1.0 + d)

    eq = jnp.concatenate([i0, i1], axis=0)
    ohi = lax.broadcasted_iota(jnp.int32, (K * T, E), 1)
    oh = (eq == ohi).astype(jnp.float32)
    tri = (lax.broadcasted_iota(jnp.int32, (PC, PC), 0)
           > lax.broadcasted_iota(jnp.int32, (PC, PC), 1)).astype(jnp.float32)
    carry = jnp.zeros((1, E), jnp.float32)
    parts = []
    for g in range(NROW):
        blk = oh[g * PC:(g + 1) * PC, :]
        wexcl = lax.dot_general(tri, blk, (((1,), (0,)), ((), ())),
                                preferred_element_type=jnp.float32)
        parts.append(wexcl + carry)
        carry = carry + jnp.sum(blk, axis=0, keepdims=True)
    excl = jnp.concatenate(parts, axis=0)

    counts = carry.astype(jnp.int32)
    pcounts = ((counts + (BT - 1)) // BT) * BT
    tri8 = (lax.broadcasted_iota(jnp.int32, (E, E), 0)
            < lax.broadcasted_iota(jnp.int32, (E, E), 1)).astype(jnp.float32)
    po = lax.dot_general(pcounts.astype(jnp.float32), tri8,
                         (((1,), (0,)), ((), ())),
                         preferred_element_type=jnp.float32)
    rank = jnp.sum(excl * oh, axis=1, keepdims=True)
    offs = lax.dot_general(oh, po.reshape(E, 1), (((1,), (0,)), ((), ())),
                           preferred_element_type=jnp.float32)
    pos = (rank + offs).astype(jnp.int32)
    pos_ref[...] = pos.reshape(NROW, PC)

    wq = jnp.concatenate([w0, w1], axis=0).reshape(NROW, PC, 1)
    wk_ref[...] = jnp.broadcast_to(wq, (NROW, PC, WREP))

    po_col = po.astype(jnp.int32).reshape(E, 1)
    biota = lax.broadcasted_iota(jnp.int32, (E, NB), 1) * BT
    beT = jnp.sum((biota >= po_col).astype(jnp.int32), axis=0, keepdims=True) - 1
    prevT = jnp.concatenate(
        [jnp.full((1, 1), -1, jnp.int32), beT[:, :-1]], axis=1)
    isfT = (beT != prevT).astype(jnp.int32)
    triNB = (lax.broadcasted_iota(jnp.int32, (NB, NB), 0)
             <= lax.broadcasted_iota(jnp.int32, (NB, NB), 1)).astype(jnp.float32)
    segT = lax.dot_general(isfT.astype(jnp.float32), triNB,
                           (((1,), (0,)), ((), ())),
                           preferred_element_type=jnp.float32).astype(jnp.int32) - 1
    slT = segT % 2
    beB = jnp.broadcast_to(beT, (NB, NB))
    beC = beT.reshape(NB, 1)
    q = jnp.where(beB > beC, beB, E)
    nxtT = jnp.min(q, axis=1, keepdims=True).reshape(1, NB)
    hnT = (nxtT < E).astype(jnp.int32)
    nxtT = jnp.minimum(nxtT, E - 1)
    pf_ref[...] = jnp.concatenate([beT, isfT, slT, nxtT, hnT], axis=0)


def _gate_dispatch(x2d, wg, interpret=False):
    return pl.pallas_call(
        _gate_dispatch_body,
        out_shape=[
            jax.ShapeDtypeStruct((NROW, PC), jnp.int32),
            jax.ShapeDtypeStruct((NROW, PC, WREP), jnp.float32),
            jax.ShapeDtypeStruct((5, NB), jnp.int32),
        ],
        interpret=interpret,
    )(x2d, wg)


def _expert_mlp_body(pf_ref, xs_ref, w1_hbm, b1_ref, w2_hbm, b2_ref, rw_ref,
                     ys_ref, w1b, w2b, sems):
    b = pl.program_id(0)
    e = pf_ref[0, b]
    isf = pf_ref[1, b]
    s = pf_ref[2, b]
    ne = pf_ref[3, b]
    hn = pf_ref[4, b]

    def start_copy(slot, expert):
        pltpu.make_async_copy(w1_hbm.at[expert], w1b.at[slot],
                              sems.at[slot, 0]).start()
        pltpu.make_async_copy(w2_hbm.at[expert], w2b.at[slot],
                              sems.at[slot, 1]).start()

    @pl.when(b == 0)
    def _():
        start_copy(s, e)

    @pl.when(isf == 1)
    def _():
        pltpu.make_async_copy(w1_hbm.at[e], w1b.at[s], sems.at[s, 0]).wait()
        pltpu.make_async_copy(w2_hbm.at[e], w2b.at[s], sems.at[s, 1]).wait()

        @pl.when(hn == 1)
        def _():
            start_copy(1 - s, ne)

    h = lax.dot_general(xs_ref[...], w1b[s], (((1,), (0,)), ((), ())),
                        preferred_element_type=jnp.float32)
    h = h + b1_ref[e]
    h = 0.5 * h * (1.0 + lax.erf(h * 0.7071067811865476))
    y = lax.dot_general(h, w2b[s], (((1,), (0,)), ((), ())),
                        preferred_element_type=jnp.float32)
    y = y + b2_ref[e]
    ys_ref[...] = y * rw_ref[:, :1]


def _expert_mlp(pf, xs, w1, b1, w2, b2, rw, interpret=False):
    grid_spec = pltpu.PrefetchScalarGridSpec(
        num_scalar_prefetch=1,
        grid=(NB,),
        in_specs=[
            pl.BlockSpec((BT, D), lambda b, pf: (b, 0)),
            pl.BlockSpec(memory_space=pltpu.HBM),
            pl.BlockSpec((E, 1, F), lambda b, pf: (0, 0, 0)),
            pl.BlockSpec(memory_space=pltpu.HBM),
            pl.BlockSpec((E, 1, D), lambda b, pf: (0, 0, 0)),
            pl.BlockSpec((BT, WREP), lambda b, pf: (b, 0)),
        ],
        out_specs=pl.BlockSpec((BT, D), lambda b, pf: (b, 0)),
        scratch_shapes=[
            pltpu.VMEM((2, D, F), jnp.float32),
            pltpu.VMEM((2, F, D), jnp.float32),
            pltpu.SemaphoreType.DMA((2, 2)),
        ],
    )
    return pl.pallas_call(
        _expert_mlp_body,
        grid_spec=grid_spec,
        out_shape=jax.ShapeDtypeStruct((NP, D), jnp.float32),
        compiler_params=pltpu.CompilerParams(
            dimension_semantics=("parallel",)),
        interpret=interpret,
    )(pf, xs, w1, b1.reshape(E, 1, F), w2, b2.reshape(E, 1, D), rw)


def _sc_mesh():
    return plsc.VectorSubcoreMesh(core_axis_name="c", subcore_axis_name="s")


def _scatter_dispatch(x2d, posr, wkr):
    @functools.partial(
        pl.kernel,
        mesh=_sc_mesh(),
        out_type=[
            jax.ShapeDtypeStruct((NP, D), jnp.float32),
            jax.ShapeDtypeStruct((NP, WREP), jnp.float32),
        ],
        scratch_types=[
            pltpu.VMEM((PC, D), jnp.float32),
            pltpu.VMEM((PC,), jnp.int32),
            pltpu.VMEM((PC, WREP), jnp.float32),
            pltpu.SemaphoreType.DMA,
            pltpu.SemaphoreType.DMA,
        ],
    )
    def k(x_hbm, pos_hbm, wk_hbm, xs_hbm, rw_hbm, xbuf, idx_v, wbuf, s1, s2):
        r = lax.axis_index("s") * 2 + lax.axis_index("c")
        t0 = (r % (NROW // K)) * PC
        pltpu.sync_copy(x_hbm.at[pl.ds(t0, PC)], xbuf)
        pltpu.sync_copy(pos_hbm.at[r], idx_v)
        pltpu.sync_copy(wk_hbm.at[r], wbuf)
        cp1 = pltpu.async_copy(xbuf, xs_hbm.at[idx_v], s1)
        cp2 = pltpu.async_copy(wbuf, rw_hbm.at[idx_v], s2)
        cp1.wait()
        cp2.wait()

    return k(x2d, posr, wkr)


def _combine(ys, posr):
    tpw = T // 32

    @functools.partial(
        pl.kernel,
        mesh=_sc_mesh(),
        out_type=jax.ShapeDtypeStruct((T, D), jnp.float32),
        scratch_types=[
            pltpu.VMEM((tpw, D), jnp.float32),
            pltpu.VMEM((tpw, D), jnp.float32),
            pltpu.VMEM((tpw,), jnp.int32),
            pltpu.VMEM((tpw,), jnp.int32),
            pltpu.SemaphoreType.DMA,
        ],
    )
    def k(ys_hbm, pos_hbm, out_hbm, a0, a1, idx0, idx1, sem):
        w = lax.axis_index("s") * 2 + lax.axis_index("c")
        tok0 = w * tpw
        r0 = w // 2
        off = (w % 2) * tpw
        pltpu.sync_copy(pos_hbm.at[r0, pl.ds(off, tpw)], idx0)
        pltpu.sync_copy(pos_hbm.at[r0 + NROW // K, pl.ds(off, tpw)], idx1)
        cp0 = pltpu.async_copy(ys_hbm.at[idx0], a0, sem)
        cp1 = pltpu.async_copy(ys_hbm.at[idx1], a1, sem)
        cp0.wait()
        cp1.wait()

        def row(i, _):
            for j in range(D // 16):
                cs = pl.ds(j * 16, 16)
                a0[i, cs] = a0[i, cs] + a1[i, cs]
            return 0
        lax.fori_loop(0, tpw, row, 0)
        pltpu.sync_copy(a0, out_hbm.at[pl.ds(tok0, tpw)])

    return k(ys, posr)


def kernel(x, Wg, W1, b1, W2, b2):
    x2d = x.reshape(T, D)
    posr, wkr, pf = _gate_dispatch(x2d, Wg)
    xs, rw = _scatter_dispatch(x2d, posr, wkr)
    ys = _expert_mlp(pf, xs, W1, b1, W2, b2, rw)
    out = _combine(ys, posr)
    return out.reshape(1, T, D)


# speedup vs baseline: 1.0897x; 1.0044x over previous
import functools

import jax
import jax.numpy as jnp
from jax import lax
from jax.experimental import pallas as pl
from jax.experimental.pallas import tpu as pltpu
from jax.experimental.pallas import tpu_sc as plsc

E = 8
K = 2
T = 2048
D = 768
F = 1024
BT = 256
NB = K * T // BT + E
NP = NB * BT
NROW = 32
PC = K * T // NROW
WREP = 128


def _gate_dispatch_body(x_ref, wg_ref, pos_ref, wk_ref, pf_ref):
    xg = x_ref[...]
    logits = lax.dot_general(xg, wg_ref[...], (((1,), (0,)), ((), ())),
                             preferred_element_type=jnp.float32)
    ei = lax.broadcasted_iota(jnp.int32, (T, E), 1)
    m0 = jnp.max(logits, axis=1, keepdims=True)
    i0 = jnp.min(jnp.where(logits == m0, ei, E), axis=1, keepdims=True)
    l2 = jnp.where(ei == i0, -jnp.inf, logits)
    m1 = jnp.max(l2, axis=1, keepdims=True)
    i1 = jnp.min(jnp.where(l2 == m1, ei, E), axis=1, keepdims=True)
    d = jnp.exp(m1 - m0)
    w0 = 1.0 / (1.0 + d)
    w1 = d / (1.0 + d)

    eq = jnp.concatenate([i0, i1], axis=0)
    ohi = lax.broadcasted_iota(jnp.int32, (K * T, E), 1)
    oh = (eq == ohi).astype(jnp.float32)
    tri = (lax.broadcasted_iota(jnp.int32, (PC, PC), 0)
           > lax.broadcasted_iota(jnp.int32, (PC, PC), 1)).astype(jnp.float32)
    carry = jnp.zeros((1, E), jnp.float32)
    parts = []
    for g in range(NROW):
        blk = oh[g * PC:(g + 1) * PC, :]
        wexcl = lax.dot_general(tri, blk, (((1,), (0,)), ((), ())),
                                preferred_element_type=jnp.float32)
        parts.append(wexcl + carry)
        carry = carry + jnp.sum(blk, axis=0, keepdims=True)
    excl = jnp.concatenate(parts, axis=0)

    counts = carry.astype(jnp.int32)
    pcounts = ((counts + (BT - 1)) // BT) * BT
    tri8 = (lax.broadcasted_iota(jnp.int32, (E, E), 0)
            < lax.broadcasted_iota(jnp.int32, (E, E), 1)).astype(jnp.float32)
    po = lax.dot_general(pcounts.astype(jnp.float32), tri8,
                         (((1,), (0,)), ((), ())),
                         preferred_element_type=jnp.float32)
    rank = jnp.sum(excl * oh, axis=1, keepdims=True)
    offs = lax.dot_general(oh, po.reshape(E, 1), (((1,), (0,)), ((), ())),
                           preferred_element_type=jnp.float32)
    pos = (rank + offs).astype(jnp.int32)
    pos_ref[...] = pos.reshape(NROW, PC)

    wq = jnp.concatenate([w0, w1], axis=0).reshape(NROW, PC, 1)
    wk_ref[...] = jnp.broadcast_to(wq, (NROW, PC, WREP))

    po_col = po.astype(jnp.int32).reshape(E, 1)
    biota = lax.broadcasted_iota(jnp.int32, (E, NB), 1) * BT
    beT = jnp.sum((biota >= po_col).astype(jnp.int32), axis=0, keepdims=True) - 1
    prevT = jnp.concatenate(
        [jnp.full((1, 1), -1, jnp.int32), beT[:, :-1]], axis=1)
    isfT = (beT != prevT).astype(jnp.int32)
    triNB = (lax.broadcasted_iota(jnp.int32, (NB, NB), 0)
             <= lax.broadcasted_iota(jnp.int32, (NB, NB), 1)).astype(jnp.float32)
    segT = lax.dot_general(isfT.astype(jnp.float32), triNB,
                           (((1,), (0,)), ((), ())),
                           preferred_element_type=jnp.float32).astype(jnp.int32) - 1
    slT = segT % 2
    beB = jnp.broadcast_to(beT, (NB, NB))
    beC = beT.reshape(NB, 1)
    q = jnp.where(beB > beC, beB, E)
    nxtT = jnp.min(q, axis=1, keepdims=True).reshape(1, NB)
    hnT = (nxtT < E).astype(jnp.int32)
    nxtT = jnp.minimum(nxtT, E - 1)
    pf_ref[...] = jnp.concatenate([beT, isfT, slT, nxtT, hnT], axis=0)


def _gate_dispatch(x2d, wg, interpret=False):
    return pl.pallas_call(
        _gate_dispatch_body,
        out_shape=[
            jax.ShapeDtypeStruct((NROW, PC), jnp.int32),
            jax.ShapeDtypeStruct((NROW, PC, WREP), jnp.float32),
            jax.ShapeDtypeStruct((5, NB), jnp.int32),
        ],
        interpret=interpret,
    )(x2d, wg)


def _expert_mlp_body(pf_ref, xs_hbm, w1_hbm, b1_ref, w2_hbm, b2_ref, rw_hbm,
                     ys_hbm, w1b, w2b, sems, xsb, rwb, ysb, isem, osem):
    b = pl.program_id(0)
    e = pf_ref[0, b]
    isf = pf_ref[1, b]
    s = pf_ref[2, b]
    ne = pf_ref[3, b]
    hn = pf_ref[4, b]

    def start_w(slot, expert):
        pltpu.make_async_copy(w1_hbm.at[expert], w1b.at[slot],
                              sems.at[slot, 0]).start()
        pltpu.make_async_copy(w2_hbm.at[expert], w2b.at[slot],
                              sems.at[slot, 1]).start()

    def start_in(slot, blk):
        pltpu.make_async_copy(xs_hbm.at[pl.ds(blk * BT, BT)], xsb.at[slot],
                              isem.at[slot, 0]).start()
        pltpu.make_async_copy(rw_hbm.at[pl.ds(blk * BT, BT)], rwb.at[slot],
                              isem.at[slot, 1]).start()

    def wait_in(slot):
        pltpu.make_async_copy(xs_hbm.at[pl.ds(0, BT)], xsb.at[slot],
                              isem.at[slot, 0]).wait()
        pltpu.make_async_copy(rw_hbm.at[pl.ds(0, BT)], rwb.at[slot],
                              isem.at[slot, 1]).wait()

    def start_out(slot, blk):
        pltpu.make_async_copy(ysb.at[slot], ys_hbm.at[pl.ds(blk * BT, BT)],
                              osem.at[slot]).start()

    def wait_out(slot):
        pltpu.make_async_copy(ysb.at[slot], ys_hbm.at[pl.ds(0, BT)],
                              osem.at[slot]).wait()

    sl = b % 2

    @pl.when(b == 0)
    def _():
        start_w(s, e)
        start_in(0, 0)

    @pl.when(b + 1 < NB)
    def _():
        start_in((b + 1) % 2, b + 1)

    @pl.when(isf == 1)
    def _():
        pltpu.make_async_copy(w1_hbm.at[e], w1b.at[s], sems.at[s, 0]).wait()
        pltpu.make_async_copy(w2_hbm.at[e], w2b.at[s], sems.at[s, 1]).wait()

        @pl.when(hn == 1)
        def _():
            start_w(1 - s, ne)

    wait_in(sl)

    @pl.when(b >= 2)
    def _():
        wait_out(sl)

    h = lax.dot_general(xsb[sl], w1b[s], (((1,), (0,)), ((), ())),
                        preferred_element_type=jnp.float32)
    h = h + b1_ref[e]
    h = 0.5 * h * (1.0 + lax.erf(h * 0.7071067811865476))
    y = lax.dot_general(h, w2b[s], (((1,), (0,)), ((), ())),
                        preferred_element_type=jnp.float32)
    y = y + b2_ref[e]
    ysb[sl] = y * rwb[sl][:, :1]
    start_out(sl, b)

    @pl.when(b == NB - 1)
    def _():
        wait_out(1 - sl)
        wait_out(sl)


def _expert_mlp(pf, xs, w1, b1, w2, b2, rw, interpret=False):
    grid_spec = pltpu.PrefetchScalarGridSpec(
        num_scalar_prefetch=1,
        grid=(NB,),
        in_specs=[
            pl.BlockSpec(memory_space=pltpu.HBM),
            pl.BlockSpec(memory_space=pltpu.HBM),
            pl.BlockSpec((E, 1, F), lambda b, pf: (0, 0, 0)),
            pl.BlockSpec(memory_space=pltpu.HBM),
            pl.BlockSpec((E, 1, D), lambda b, pf: (0, 0, 0)),
            pl.BlockSpec(memory_space=pltpu.HBM),
        ],
        out_specs=pl.BlockSpec(memory_space=pltpu.HBM),
        scratch_shapes=[
            pltpu.VMEM((2, D, F), jnp.float32),
            pltpu.VMEM((2, F, D), jnp.float32),
            pltpu.SemaphoreType.DMA((2, 2)),
            pltpu.VMEM((2, BT, D), jnp.float32),
            pltpu.VMEM((2, BT, WREP), jnp.float32),
            pltpu.VMEM((2, BT, D), jnp.float32),
            pltpu.SemaphoreType.DMA((2, 2)),
            pltpu.SemaphoreType.DMA((2,)),
        ],
    )
    return pl.pallas_call(
        _expert_mlp_body,
        grid_spec=grid_spec,
        out_shape=jax.ShapeDtypeStruct((NP, D), jnp.float32),
        compiler_params=pltpu.CompilerParams(
            dimension_semantics=("parallel",)),
        interpret=interpret,
    )(pf, xs, w1, b1.reshape(E, 1, F), w2, b2.reshape(E, 1, D), rw)


def _sc_mesh():
    return plsc.VectorSubcoreMesh(core_axis_name="c", subcore_axis_name="s")


def _scatter_dispatch(x2d, posr, wkr):
    @functools.partial(
        pl.kernel,
        mesh=_sc_mesh(),
        out_type=[
            jax.ShapeDtypeStruct((NP, D), jnp.float32),
            jax.ShapeDtypeStruct((NP, WREP), jnp.float32),
        ],
        scratch_types=[
            pltpu.VMEM((PC, D), jnp.float32),
            pltpu.VMEM((PC,), jnp.int32),
            pltpu.VMEM((PC, WREP), jnp.float32),
            pltpu.SemaphoreType.DMA,
            pltpu.SemaphoreType.DMA,
        ],
    )
    def k(x_hbm, pos_hbm, wk_hbm, xs_hbm, rw_hbm, xbuf, idx_v, wbuf, s1, s2):
        r = lax.axis_index("s") * 2 + lax.axis_index("c")
        t0 = (r % (NROW // K)) * PC
        pltpu.sync_copy(x_hbm.at[pl.ds(t0, PC)], xbuf)
        pltpu.sync_copy(pos_hbm.at[r], idx_v)
        pltpu.sync_copy(wk_hbm.at[r], wbuf)
        cp1 = pltpu.async_copy(xbuf, xs_hbm.at[idx_v], s1)
        cp2 = pltpu.async_copy(wbuf, rw_hbm.at[idx_v], s2)
        cp1.wait()
        cp2.wait()

    return k(x2d, posr, wkr)


def _combine(ys, posr):
    tpw = T // 32

    @functools.partial(
        pl.kernel,
        mesh=_sc_mesh(),
        out_type=jax.ShapeDtypeStruct((T, D), jnp.float32),
        scratch_types=[
            pltpu.VMEM((tpw, D), jnp.float32),
            pltpu.VMEM((tpw, D), jnp.float32),
            pltpu.VMEM((tpw,), jnp.int32),
            pltpu.VMEM((tpw,), jnp.int32),
            pltpu.SemaphoreType.DMA,
        ],
    )
    def k(ys_hbm, pos_hbm, out_hbm, a0, a1, idx0, idx1, sem):
        w = lax.axis_index("s") * 2 + lax.axis_index("c")
        tok0 = w * tpw
        r0 = w // 2
        off = (w % 2) * tpw
        pltpu.sync_copy(pos_hbm.at[r0, pl.ds(off, tpw)], idx0)
        pltpu.sync_copy(pos_hbm.at[r0 + NROW // K, pl.ds(off, tpw)], idx1)
        cp0 = pltpu.async_copy(ys_hbm.at[idx0], a0, sem)
        cp1 = pltpu.async_copy(ys_hbm.at[idx1], a1, sem)
        cp0.wait()
        cp1.wait()

        def row(i, _):
            for j in range(D // 16):
                cs = pl.ds(j * 16, 16)
                a0[i, cs] = a0[i, cs] + a1[i, cs]
            return 0
        lax.fori_loop(0, tpw, row, 0)
        pltpu.sync_copy(a0, out_hbm.at[pl.ds(tok0, tpw)])

    return k(ys, posr)


def kernel(x, Wg, W1, b1, W2, b2):
    x2d = x.reshape(T, D)
    posr, wkr, pf = _gate_dispatch(x2d, Wg)
    xs, rw = _scatter_dispatch(x2d, posr, wkr)
    ys = _expert_mlp(pf, xs, W1, b1, W2, b2, rw)
    out = _combine(ys, posr)
    return out.reshape(1, T, D)
